# speedup vs baseline: 1.2461x; 1.2461x over previous
import functools

import jax
import jax.numpy as jnp
from jax import lax
from jax.experimental import pallas as pl
from jax.experimental.pallas import tpu as pltpu
from jax.experimental.pallas import tpu_sc as plsc


@functools.lru_cache(maxsize=None)
def _build_gather(V, D, N):
    info = plsc.get_sparse_core_info()
    NC, NS = info.num_cores, info.num_subcores
    NW = NC * NS
    assert N % NW == 0
    b_per_w = N // NW
    C = 32
    assert b_per_w % C == 0
    n_chunks = b_per_w // C

    mesh = plsc.VectorSubcoreMesh(core_axis_name="c", subcore_axis_name="s")

    @functools.partial(
        pl.kernel,
        mesh=mesh,
        out_type=jax.ShapeDtypeStruct((NW, b_per_w, D), jnp.float32),
        scratch_types=[
            pltpu.VMEM((n_chunks, C), jnp.int32),
            pltpu.VMEM((C, D), jnp.float32),
            pltpu.SemaphoreType.DMA,
        ],
    )
    def gather_k(table_hbm, idx_hbm, out_hbm, idx_v, rows_v, sem):
        wid = lax.axis_index("s") * NC + lax.axis_index("c")
        pltpu.sync_copy(idx_hbm.at[wid], idx_v)

        def body(g, carry):
            pltpu.async_copy(table_hbm.at[idx_v.at[g]], rows_v, sem).wait()
            pltpu.sync_copy(rows_v, out_hbm.at[wid, pl.ds(g * C, C)])
            return carry

        lax.fori_loop(0, n_chunks, body, 0)

    return gather_k, NW, n_chunks, C


def kernel(input_ids, attention_mask, table):
    B, L = input_ids.shape
    V, D = table.shape
    N = B * L
    gather_k, NW, n_chunks, C = _build_gather(V, D, N)
    idx3 = input_ids.reshape(NW, n_chunks, C).astype(jnp.int32)
    out = gather_k(table, idx3)
    return out.reshape(B, L, D)


# speedup vs baseline: 1.8004x; 1.4448x over previous
import functools

import jax
import jax.numpy as jnp
from jax import lax
from jax.experimental import pallas as pl
from jax.experimental.pallas import tpu as pltpu
from jax.experimental.pallas import tpu_sc as plsc


@functools.lru_cache(maxsize=None)
def _build_gather(V, D, N):
    info = plsc.get_sparse_core_info()
    NC, NS = info.num_cores, info.num_subcores
    NW = NC * NS
    assert N % NW == 0
    b_per_w = N // NW
    C = 32
    assert b_per_w % C == 0
    n_chunks = b_per_w // C

    nbuf = 4
    K = 2
    n_outer = n_chunks // nbuf
    assert n_chunks % nbuf == 0 and n_outer >= 2

    mesh = plsc.VectorSubcoreMesh(core_axis_name="c", subcore_axis_name="s")

    @functools.partial(
        pl.kernel,
        mesh=mesh,
        out_type=jax.ShapeDtypeStruct((NW, b_per_w, D), jnp.float32),
        scratch_types=[
            pltpu.VMEM((n_chunks, C), jnp.int32),
            pltpu.VMEM((nbuf, C, D), jnp.float32),
        ]
        + [pltpu.SemaphoreType.DMA] * (2 * nbuf),
    )
    def gather_k(table_hbm, idx_hbm, out_hbm, idx_v, rows_v, *sems):
        gsem, psem = sems[:nbuf], sems[nbuf:]
        wid = lax.axis_index("s") * NC + lax.axis_index("c")
        pltpu.sync_copy(idx_hbm.at[wid], idx_v)

        def start_gather(g, b):
            pltpu.async_copy(table_hbm.at[idx_v.at[g]], rows_v.at[b], gsem[b])

        def wait_gather(g, b):
            pltpu.make_async_copy(
                table_hbm.at[idx_v.at[g]], rows_v.at[b], gsem[b]
            ).wait()

        def start_put(g, b):
            pltpu.async_copy(rows_v.at[b], out_hbm.at[wid, pl.ds(g * C, C)], psem[b])

        def wait_put(g, b):
            pltpu.make_async_copy(
                rows_v.at[b], out_hbm.at[wid, pl.ds(g * C, C)], psem[b]
            ).wait()

        for b in range(K):
            start_gather(b, b)

        for b in range(nbuf):
            wait_gather(b, b)
            start_put(b, b)
            nb = (b + K) % nbuf
            if b >= K:
                wait_put(b - K, nb)
            start_gather(b + K, nb)

        def body(o, carry):
            g0 = o * nbuf
            for b in range(nbuf):
                g = g0 + b
                wait_gather(g, b)
                start_put(g, b)
                nb = (b + K) % nbuf
                wait_put(g - K, nb)
                start_gather(g + K, nb)
            return carry

        lax.fori_loop(1, n_outer - 1, body, 0)

        g0 = (n_outer - 1) * nbuf
        for b in range(nbuf):
            g = g0 + b
            wait_gather(g, b)
            start_put(g, b)
            if b < K:
                nb = (b + K) % nbuf
                wait_put(g - K, nb)
                start_gather(g + K, nb)

        for b in range(nbuf):
            wait_put(g0 + b, b)

    return gather_k, NW, n_chunks, C


def kernel(input_ids, attention_mask, table):
    B, L = input_ids.shape
    V, D = table.shape
    N = B * L
    gather_k, NW, n_chunks, C = _build_gather(V, D, N)
    idx3 = input_ids.reshape(NW, n_chunks, C).astype(jnp.int32)
    out = gather_k(table, idx3)
    return out.reshape(B, L, D)


# speedup vs baseline: 1.8087x; 1.0046x over previous
import functools

import jax
import jax.numpy as jnp
from jax import lax
from jax.experimental import pallas as pl
from jax.experimental.pallas import tpu as pltpu
from jax.experimental.pallas import tpu_sc as plsc


@functools.lru_cache(maxsize=None)
def _build_gather(V, D, N):
    info = plsc.get_sparse_core_info()
    NC, NS = info.num_cores, info.num_subcores
    NW = NC * NS
    assert N % NW == 0
    b_per_w = N // NW
    C = 40
    assert b_per_w % C == 0
    n_chunks = b_per_w // C

    nbuf = 4
    K = 2
    n_outer = n_chunks // nbuf
    assert n_chunks % nbuf == 0 and n_outer >= 2

    mesh = plsc.VectorSubcoreMesh(core_axis_name="c", subcore_axis_name="s")

    @functools.partial(
        pl.kernel,
        mesh=mesh,
        out_type=jax.ShapeDtypeStruct((NW, b_per_w, D), jnp.float32),
        scratch_types=[
            pltpu.VMEM((n_chunks, C), jnp.int32),
            pltpu.VMEM((nbuf, C, D), jnp.float32),
        ]
        + [pltpu.SemaphoreType.DMA] * (2 * nbuf),
    )
    def gather_k(table_hbm, idx_hbm, out_hbm, idx_v, rows_v, *sems):
        gsem, psem = sems[:nbuf], sems[nbuf:]
        wid = lax.axis_index("s") * NC + lax.axis_index("c")
        pltpu.sync_copy(idx_hbm.at[wid], idx_v)

        def start_gather(g, b):
            pltpu.async_copy(table_hbm.at[idx_v.at[g]], rows_v.at[b], gsem[b])

        def wait_gather(g, b):
            pltpu.make_async_copy(
                table_hbm.at[idx_v.at[g]], rows_v.at[b], gsem[b]
            ).wait()

        def start_put(g, b):
            pltpu.async_copy(rows_v.at[b], out_hbm.at[wid, pl.ds(g * C, C)], psem[b])

        def wait_put(g, b):
            pltpu.make_async_copy(
                rows_v.at[b], out_hbm.at[wid, pl.ds(g * C, C)], psem[b]
            ).wait()

        for b in range(K):
            start_gather(b, b)

        for b in range(nbuf):
            wait_gather(b, b)
            start_put(b, b)
            nb = (b + K) % nbuf
            if b >= K:
                wait_put(b - K, nb)
            start_gather(b + K, nb)

        def body(o, carry):
            g0 = o * nbuf
            for b in range(nbuf):
                g = g0 + b
                wait_gather(g, b)
                start_put(g, b)
                nb = (b + K) % nbuf
                wait_put(g - K, nb)
                start_gather(g + K, nb)
            return carry

        lax.fori_loop(1, n_outer - 1, body, 0)

        g0 = (n_outer - 1) * nbuf
        for b in range(nbuf):
            g = g0 + b
            wait_gather(g, b)
            start_put(g, b)
            if b < K:
                nb = (b + K) % nbuf
                wait_put(g - K, nb)
                start_gather(g + K, nb)

        for b in range(nbuf):
            wait_put(g0 + b, b)

    return gather_k, NW, n_chunks, C


def kernel(input_ids, attention_mask, table):
    B, L = input_ids.shape
    V, D = table.shape
    N = B * L
    gather_k, NW, n_chunks, C = _build_gather(V, D, N)
    idx3 = input_ids.reshape(NW, n_chunks, C).astype(jnp.int32)
    out = gather_k(table, idx3)
    return out.reshape(B, L, D)


# speedup vs baseline: 1.8112x; 1.0014x over previous
import functools

import jax
import jax.numpy as jnp
from jax import lax
from jax.experimental import pallas as pl
from jax.experimental.pallas import tpu as pltpu
from jax.experimental.pallas import tpu_sc as plsc


@functools.lru_cache(maxsize=None)
def _build_gather(V, D, N):
    info = plsc.get_sparse_core_info()
    NC, NS = info.num_cores, info.num_subcores
    NW = NC * NS
    assert N % NW == 0
    b_per_w = N // NW
    C = 40
    assert b_per_w % C == 0
    n_chunks = b_per_w // C

    nbuf = 4
    K = 2
    n_outer = n_chunks // nbuf
    assert n_chunks % nbuf == 0 and n_outer >= 2

    mesh = plsc.VectorSubcoreMesh(core_axis_name="c", subcore_axis_name="s")

    @functools.partial(
        pl.kernel,
        mesh=mesh,
        out_type=jax.ShapeDtypeStruct((NW, b_per_w, D), jnp.float32),
        scratch_types=[
            pltpu.VMEM((n_chunks, C), jnp.int32),
            pltpu.VMEM((nbuf, C, D), jnp.float32),
        ]
        + [pltpu.SemaphoreType.DMA] * (2 * nbuf),
    )
    def gather_k(table_hbm, idx_hbm, out_hbm, idx_v, rows_v, *sems):
        gsem, psem = sems[:nbuf], sems[nbuf:]
        wid = lax.axis_index("s") * NC + lax.axis_index("c")
        pltpu.sync_copy(idx_hbm.at[wid], idx_v)

        def start_gather(g, b):
            pltpu.async_copy(table_hbm.at[idx_v.at[g]], rows_v.at[b], gsem[b])

        def wait_gather(g, b):
            pltpu.make_async_copy(
                table_hbm.at[idx_v.at[g]], rows_v.at[b], gsem[b]
            ).wait()

        def start_put(g, b):
            pltpu.async_copy(rows_v.at[b], out_hbm.at[wid, pl.ds(g * C, C)], psem[b])

        def wait_put(g, b):
            pltpu.make_async_copy(
                rows_v.at[b], out_hbm.at[wid, pl.ds(g * C, C)], psem[b]
            ).wait()

        for b in range(K):
            start_gather(b, b)

        for b in range(nbuf):
            wait_gather(b, b)
            start_put(b, b)
            nb = (b + K) % nbuf
            if b >= K:
                wait_put(b - K, nb)
            start_gather(b + K, nb)

        def body(o, carry):
            g0 = o * nbuf
            for b in range(nbuf):
                g = g0 + b
                nb = (b + K) % nbuf
                wait_put(g - K, nb)
                start_gather(g + K, nb)
                wait_gather(g, b)
                start_put(g, b)
            return carry

        lax.fori_loop(1, n_outer - 1, body, 0)

        g0 = (n_outer - 1) * nbuf
        for b in range(nbuf):
            g = g0 + b
            wait_gather(g, b)
            start_put(g, b)
            if b < K:
                nb = (b + K) % nbuf
                wait_put(g - K, nb)
                start_gather(g + K, nb)

        for b in range(nbuf):
            wait_put(g0 + b, b)

    return gather_k, NW, n_chunks, C


def kernel(input_ids, attention_mask, table):
    B, L = input_ids.shape
    V, D = table.shape
    N = B * L
    gather_k, NW, n_chunks, C = _build_gather(V, D, N)
    idx3 = input_ids.reshape(NW, n_chunks, C).astype(jnp.int32)
    out = gather_k(table, idx3)
    return out.reshape(B, L, D)


# speedup vs baseline: 1.8154x; 1.0023x over previous
import functools

import jax
import jax.numpy as jnp
from jax import lax
from jax.experimental import pallas as pl
from jax.experimental.pallas import tpu as pltpu
from jax.experimental.pallas import tpu_sc as plsc


@functools.lru_cache(maxsize=None)
def _build_gather(V, D, N):
    info = plsc.get_sparse_core_info()
    NC, NS = info.num_cores, info.num_subcores
    NW = NC * NS
    assert N % NW == 0
    b_per_w = N // NW
    C = 40
    assert b_per_w % C == 0
    n_chunks = b_per_w // C

    nbuf = 5
    K = 2
    n_outer = n_chunks // nbuf
    assert n_chunks % nbuf == 0 and n_outer >= 2

    mesh = plsc.VectorSubcoreMesh(core_axis_name="c", subcore_axis_name="s")

    @functools.partial(
        pl.kernel,
        mesh=mesh,
        out_type=jax.ShapeDtypeStruct((NW, b_per_w, D), jnp.float32),
        scratch_types=[
            pltpu.VMEM((n_chunks, C), jnp.int32),
            pltpu.VMEM((nbuf, C, D), jnp.float32),
        ]
        + [pltpu.SemaphoreType.DMA] * (2 * nbuf),
    )
    def gather_k(table_hbm, idx_hbm, out_hbm, idx_v, rows_v, *sems):
        gsem, psem = sems[:nbuf], sems[nbuf:]
        wid = lax.axis_index("s") * NC + lax.axis_index("c")
        pltpu.sync_copy(idx_hbm.at[wid], idx_v)

        def start_gather(g, b):
            pltpu.async_copy(table_hbm.at[idx_v.at[g]], rows_v.at[b], gsem[b])

        def wait_gather(g, b):
            pltpu.make_async_copy(
                table_hbm.at[idx_v.at[g]], rows_v.at[b], gsem[b]
            ).wait()

        def start_put(g, b):
            pltpu.async_copy(rows_v.at[b], out_hbm.at[wid, pl.ds(g * C, C)], psem[b])

        def wait_put(g, b):
            pltpu.make_async_copy(
                rows_v.at[b], out_hbm.at[wid, pl.ds(g * C, C)], psem[b]
            ).wait()

        for b in range(K):
            start_gather(b, b)

        for b in range(nbuf):
            wait_gather(b, b)
            start_put(b, b)
            nb = (b + K) % nbuf
            if b + K - nbuf >= 0:
                wait_put(b + K - nbuf, nb)
            start_gather(b + K, nb)

        def body(o, carry):
            g0 = o * nbuf
            for b in range(nbuf):
                g = g0 + b
                nb = (b + K) % nbuf
                wait_put(g + K - nbuf, nb)
                start_gather(g + K, nb)
                wait_gather(g, b)
                start_put(g, b)
            return carry

        lax.fori_loop(1, n_outer - 1, body, 0)

        g0 = (n_outer - 1) * nbuf
        for b in range(nbuf):
            g = g0 + b
            wait_gather(g, b)
            start_put(g, b)
            if b < nbuf - K:
                nb = (b + K) % nbuf
                wait_put(g + K - nbuf, nb)
                start_gather(g + K, nb)

        for b in range(nbuf):
            wait_put(g0 + b, b)

    return gather_k, NW, n_chunks, C


def kernel(input_ids, attention_mask, table):
    B, L = input_ids.shape
    V, D = table.shape
    N = B * L
    gather_k, NW, n_chunks, C = _build_gather(V, D, N)
    idx3 = input_ids.reshape(NW, n_chunks, C).astype(jnp.int32)
    out = gather_k(table, idx3)
    return out.reshape(B, L, D)
